# Initial kernel scaffold; baseline (speedup 1.0000x reference)
import jax
import jax.numpy as jnp
from jax.experimental import pallas as pl


def kernel(support, query):
    raise NotImplementedError("write your pallas kernel here")


# speedup vs baseline: 10.6851x; 10.6851x over previous
import functools

import jax
import jax.numpy as jnp
from jax.experimental import pallas as pl
from jax.experimental.pallas import tpu as pltpu

_N = 8192
_C = 256
_K = 16
_MT = 256


def _knn_body(s_ref, q_ref, vals_ref, idxs_ref, d_ref):
    s = s_ref[0]
    q = q_ref[0]
    r = jax.lax.dot_general(
        s, q, (((1,), (1,)), ((), ())), preferred_element_type=jnp.float32
    )
    s2 = jnp.sum(s * s, axis=1, keepdims=True)
    q2 = jnp.sum(q * q, axis=1)[None, :]
    d_ref[...] = jnp.sqrt(jnp.clip(s2 + q2 - 2.0 * r, 0.0, None))

    iota = jax.lax.broadcasted_iota(jnp.int32, d_ref.shape, 0)

    def body(k, _):
        d = d_ref[...]
        m = jnp.min(d, axis=0, keepdims=True)
        hit = d == m
        idx = jnp.min(jnp.where(hit, iota, d_ref.shape[0]), axis=0, keepdims=True)
        vals_ref[0, k, :] = m[0]
        idxs_ref[0, k, :] = idx[0]
        d_ref[...] = jnp.where(iota == idx, jnp.inf, d)
        return 0

    jax.lax.fori_loop(0, _K, body, 0, unroll=False)


@functools.partial(jax.jit, static_argnames=("interpret",))
def kernel(support, query, interpret=False):
    b, n, c = support.shape
    m = query.shape[1]
    grid = (b, m // _MT)
    vals, idxs = pl.pallas_call(
        _knn_body,
        grid=grid,
        in_specs=[
            pl.BlockSpec((1, n, c), lambda bi, mi: (bi, 0, 0)),
            pl.BlockSpec((1, _MT, c), lambda bi, mi: (bi, mi, 0)),
        ],
        out_specs=[
            pl.BlockSpec((1, _K, _MT), lambda bi, mi: (bi, 0, mi)),
            pl.BlockSpec((1, _K, _MT), lambda bi, mi: (bi, 0, mi)),
        ],
        out_shape=[
            jax.ShapeDtypeStruct((b, _K, m), jnp.float32),
            jax.ShapeDtypeStruct((b, _K, m), jnp.int32),
        ],
        scratch_shapes=[pltpu.VMEM((n, _MT), jnp.float32)],
        interpret=interpret,
    )(support, query)
    return vals, jnp.transpose(idxs, (0, 2, 1))
